# baseline (speedup 1.0000x reference)
import jax
import jax.numpy as jnp
from jax.experimental import pallas as pl
from jax.experimental.pallas import tpu as pltpu

_BM = 512


def _ff_body(x_ref, w_ref, o_ref):
    o_ref[...] = jax.lax.dot(
        x_ref[...].astype(jnp.bfloat16),
        w_ref[...].astype(jnp.bfloat16),
        preferred_element_type=jnp.float32,
    ).astype(jnp.bfloat16)


def _feature_transform(x, w):
    n = x.shape[0]
    d_out = w.shape[1]
    return pl.pallas_call(
        _ff_body,
        out_shape=jax.ShapeDtypeStruct((n, d_out), jnp.bfloat16),
    )(x, w)


def _agg_body(s_ref, b_ref, adj_ref, o_ref):
    a = adj_ref[...].astype(jnp.bfloat16)
    y = jax.lax.dot(a, s_ref[...], preferred_element_type=jnp.float32)
    y = y + b_ref[...]
    o_ref[...] = jnp.where(y >= 0, y, 0.01 * y)


def _aggregate(adj, support, b):
    n = adj.shape[0]
    d = support.shape[1]
    return pl.pallas_call(
        _agg_body,
        grid=(n // _BM,),
        in_specs=[
            pl.BlockSpec((n, d), lambda i: (0, 0)),
            pl.BlockSpec((1, d), lambda i: (0, 0)),
            pl.BlockSpec((_BM, n), lambda i: (i, 0)),
        ],
        out_specs=pl.BlockSpec((_BM, d), lambda i: (i, 0)),
        out_shape=jax.ShapeDtypeStruct((n, d), jnp.float32),
        compiler_params=pltpu.CompilerParams(
            dimension_semantics=("arbitrary",),
        ),
    )(support, b.reshape(1, d), adj)


def kernel(x, adj, W1, b1, W2, b2):
    s1 = _feature_transform(x, W1)
    h = _aggregate(adj, s1, b1)
    s2 = _feature_transform(h, W2)
    return _aggregate(adj, s2, b2)


# speedup vs baseline: 1.0853x; 1.0853x over previous
import jax
import jax.numpy as jnp
from jax.experimental import pallas as pl
from jax.experimental.pallas import tpu as pltpu

_BM = 512


def _bf16(v):
    return v.astype(jnp.bfloat16)


def _gcn_body(x_ref, w1_ref, b1_ref, w2_ref, b2_ref, adj_ref, o_ref, s_s, h_s):
    l = pl.program_id(0)
    i = pl.program_id(1)

    @pl.when((l == 0) & (i == 0))
    def _():
        s_s[...] = _bf16(jax.lax.dot(
            _bf16(x_ref[...]), _bf16(w1_ref[...]),
            preferred_element_type=jnp.float32))

    @pl.when((l == 1) & (i == 0))
    def _():
        s_s[...] = _bf16(jax.lax.dot(
            _bf16(h_s[...]), _bf16(w2_ref[...]),
            preferred_element_type=jnp.float32))

    y = jax.lax.dot(_bf16(adj_ref[...]), s_s[...],
                    preferred_element_type=jnp.float32)
    y = y + jnp.where(l == 0, b1_ref[...], b2_ref[...])
    y = jnp.where(y >= 0, y, 0.01 * y)

    @pl.when(l == 0)
    def _():
        h_s[pl.ds(i * _BM, _BM), :] = y

    @pl.when(l == 1)
    def _():
        o_ref[...] = y


def kernel(x, adj, W1, b1, W2, b2):
    n, d_in = x.shape
    d_hid = W1.shape[1]
    d_out = W2.shape[1]
    nb = n // _BM
    return pl.pallas_call(
        _gcn_body,
        grid=(2, nb),
        in_specs=[
            pl.BlockSpec((n, d_in), lambda l, i: (0, 0)),
            pl.BlockSpec((d_in, d_hid), lambda l, i: (0, 0)),
            pl.BlockSpec((1, d_hid), lambda l, i: (0, 0)),
            pl.BlockSpec((d_hid, d_out), lambda l, i: (0, 0)),
            pl.BlockSpec((1, d_out), lambda l, i: (0, 0)),
            pl.BlockSpec((_BM, n), lambda l, i: (i, 0)),
        ],
        out_specs=pl.BlockSpec((_BM, d_out), lambda l, i: (l * i, 0)),
        out_shape=jax.ShapeDtypeStruct((n, d_out), jnp.float32),
        scratch_shapes=[
            pltpu.VMEM((n, d_hid), jnp.bfloat16),
            pltpu.VMEM((n, d_hid), jnp.float32),
        ],
        compiler_params=pltpu.CompilerParams(
            dimension_semantics=("arbitrary", "arbitrary"),
        ),
    )(x, W1, b1.reshape(1, d_hid), W2, b2.reshape(1, d_out), adj)
